# baseline (speedup 1.0000x reference)
import functools

import jax
import jax.numpy as jnp
from jax import lax
from jax.experimental import pallas as pl
from jax.experimental.pallas import tpu as pltpu
from jax.experimental.pallas import tpu_sc as plsc

ALPHA = 0.1
KSTEPS = 5
NSUB = 16
NCORE = 2
LANES = 16
CHUNK = 128

_SC_PARAMS = pltpu.CompilerParams(
    needs_layout_passes=False, use_tc_tiling_on_sc=False)


def _sc_degree(n_pad, rpt, chunks):
    mesh = plsc.VectorSubcoreMesh(core_axis_name="c", subcore_axis_name="s")

    half = -(-chunks // 2)

    @functools.partial(
        pl.kernel,
        out_type=jax.ShapeDtypeStruct((NCORE, n_pad), jnp.float32),
        mesh=mesh,
        compiler_params=_SC_PARAMS,
        scratch_types=[
            pltpu.VMEM_SHARED((n_pad,), jnp.float32),
            pltpu.VMEM((CHUNK,), jnp.int32),
            pltpu.VMEM((CHUNK,), jnp.int32),
            pltpu.VMEM((CHUNK,), jnp.float32),
            pltpu.VMEM((CHUNK,), jnp.float32),
            pltpu.SemaphoreType.DMA,
            pltpu.SemaphoreType.DMA,
        ],
    )
    def deg_kernel(ei_hbm, deg_out, DEG, didx_a, didx_b, ones_t, init_t,
                   isem_a, isem_b):
        c = lax.axis_index("c")
        s = lax.axis_index("s")
        row0 = s * rpt
        ones16 = jnp.full((LANES,), 1.0, jnp.float32)
        init16 = jnp.full((LANES,), jnp.where(c == 0, 1.0, 0.0))

        def f_ones(i, carry):
            ones_t[pl.ds(i * LANES, LANES)] = ones16
            init_t[pl.ds(i * LANES, LANES)] = init16
            return carry
        lax.fori_loop(0, CHUNK // LANES, f_ones, 0)

        def f_deginit(j, carry):
            pltpu.sync_copy(init_t, DEG.at[pl.ds(row0 + j * CHUNK, CHUNK)])
            return carry
        lax.fori_loop(0, rpt // CHUNK, f_deginit, 0)
        plsc.subcore_barrier()

        w0 = s * chunks + c * half
        hc = jnp.where(c == 0, half, chunks - half)
        wlast = s * chunks + chunks - 1
        ia = pltpu.make_async_copy(ei_hbm.at[w0, 1], didx_a, isem_a)
        ia.start()

        def f_deg(i2, carry):
            c0 = w0 + 2 * i2
            ib = pltpu.make_async_copy(
                ei_hbm.at[jnp.minimum(c0 + 1, wlast), 1], didx_b, isem_b)
            ib.start()
            ia.wait()

            @pl.when(c0 < w0 + hc)
            def _():
                pltpu.sync_copy(ones_t, DEG.at[didx_a], add=True)
            ia2 = pltpu.make_async_copy(
                ei_hbm.at[jnp.minimum(c0 + 2, wlast), 1], didx_a, isem_a)
            ia2.start()
            ib.wait()

            @pl.when(c0 + 1 < w0 + hc)
            def _():
                pltpu.sync_copy(ones_t, DEG.at[didx_b], add=True)
            return carry
        lax.fori_loop(0, (half + 1) // 2, f_deg, 0)
        ia.wait()
        plsc.subcore_barrier()
        pltpu.sync_copy(DEG.at[pl.ds(row0, rpt)],
                        deg_out.at[c, pl.ds(row0, rpt)])

    return deg_kernel


def _sc_propagate(n_pad, f_half, rpt, chunks):
    mesh = plsc.VectorSubcoreMesh(core_axis_name="c", subcore_axis_name="s")

    @functools.partial(
        pl.kernel,
        out_type=jax.ShapeDtypeStruct((KSTEPS, NCORE, n_pad, f_half),
                                      jnp.float32),
        mesh=mesh,
        compiler_params=_SC_PARAMS,
        scratch_types=[
            pltpu.VMEM_SHARED((n_pad, f_half), jnp.float32),
            pltpu.VMEM_SHARED((n_pad, f_half), jnp.float32),
            pltpu.VMEM((CHUNK, f_half), jnp.float32),
            pltpu.VMEM((CHUNK, f_half), jnp.float32),
            pltpu.VMEM((CHUNK, f_half), jnp.float32),
            pltpu.VMEM((CHUNK, f_half), jnp.float32),
            pltpu.VMEM((CHUNK, f_half), jnp.float32),
            pltpu.VMEM((2, CHUNK), jnp.int32),
            pltpu.VMEM((2, CHUNK), jnp.int32),
            pltpu.VMEM((2, CHUNK), jnp.int32),
            pltpu.VMEM((rpt,), jnp.float32),
            pltpu.VMEM((rpt,), jnp.float32),
            pltpu.SemaphoreType.DMA,
            pltpu.SemaphoreType.DMA,
            pltpu.SemaphoreType.DMA,
            pltpu.SemaphoreType.DMA,
            pltpu.SemaphoreType.DMA,
            pltpu.SemaphoreType.DMA,
            pltpu.SemaphoreType.DMA,
            pltpu.SemaphoreType.DMA,
            pltpu.SemaphoreType.DMA,
            pltpu.SemaphoreType.DMA,
            pltpu.SemaphoreType.DMA,
            pltpu.SemaphoreType.DMA,
            pltpu.SemaphoreType.DMA,
            pltpu.SemaphoreType.DMA,
            pltpu.SemaphoreType.DMA,
            pltpu.SemaphoreType.DMA,
            pltpu.SemaphoreType.DMA,
        ],
    )
    def prop(x2_hbm, ei_hbm, dinv_hbm, recip_hbm, u_out,
             U, Y, ytile, ytile2, rows_0, rows_1, rows_2, idx_0, idx_1, idx_2,
             dinv_v, recip_v, gsem_0, gsem_1, gsem_2, ssem_0, ssem_1, ssem_2,
             isem_0, isem_1, isem_2, yr0, yr1,
             yw0a, yw0b, yw0c, yw1a, yw1b, yw1c):
        c = lax.axis_index("c")
        s = lax.axis_index("s")
        row0 = s * rpt
        ch0 = s * chunks
        nsub = rpt // CHUNK

        pltpu.sync_copy(dinv_hbm.at[pl.ds(row0, rpt)], dinv_v)
        pltpu.sync_copy(recip_hbm.at[pl.ds(row0, rpt)], recip_v)

        def f_x(j, carry):
            r0 = row0 + j * CHUNK
            pltpu.sync_copy(x2_hbm.at[c, pl.ds(r0, CHUNK)], ytile)

            def f_row(r, carry2):
                idx = jnp.full((LANES,), j * CHUNK + r, jnp.int32)
                dv = plsc.load_gather(dinv_v, [idx])
                for c2 in range(f_half // LANES):
                    sl = pl.ds(c2 * LANES, LANES)
                    ytile[r, sl] = ytile[r, sl] * dv
                return carry2
            lax.fori_loop(0, CHUNK, f_row, 0)
            pltpu.sync_copy(ytile, U.at[pl.ds(r0, CHUNK)])
            pltpu.sync_copy(ytile, Y.at[pl.ds(r0, CHUNK)])
            return carry
        lax.fori_loop(0, nsub, f_x, 0)
        plsc.subcore_barrier()


        rows = (rows_0, rows_1, rows_2)
        idx = (idx_0, idx_1, idx_2)
        gsem = (gsem_0, gsem_1, gsem_2)
        ssem = (ssem_0, ssem_1, ssem_2)
        isem = (isem_0, isem_1, isem_2)
        yw0 = (yw0a, yw0b, yw0c)
        yw1 = (yw1a, yw1b, yw1c)
        gd = tuple(pltpu.make_async_copy(U.at[idx[j].at[0]], rows[j], gsem[j])
                   for j in range(3))
        clast = ch0 + chunks - 1

        for k in range(1, KSTEPS + 1):
            pltpu.sync_copy(ei_hbm.at[ch0], idx_0)
            gd[0].start()
            pltpu.sync_copy(ei_hbm.at[ch0 + 1], idx_1)
            gd[1].start()

            def f_tri(i3, carry):
                c0 = ch0 + 3 * i3
                scat = []
                stage = {}
                for j in range(3):
                    if j > 0:
                        jq = j - 2 if j == 2 else 2
                        stage[jq].wait()
                        gd[jq].start()
                    gd[j].wait()
                    scat.append(pltpu.async_copy(
                        rows[j], Y.at[idx[j].at[1]], ssem[j], add=True))
                    if j > 0:
                        scat[j - 1].wait()
                        jp = j - 1
                    else:
                        jp = 2
                    cn = jnp.minimum(c0 + 2 + j, clast)
                    stage[jp] = pltpu.async_copy(ei_hbm.at[cn], idx[jp],
                                                 isem[jp])
                scat[2].wait()
                stage[1].wait()
                gd[1].start()
                return carry
            lax.fori_loop(0, chunks // 3, f_tri, 0)
            gd[0].wait()
            gd[1].wait()
            plsc.subcore_barrier()

            def scale(buf, jj):
                def f_row(r, carry2):
                    bidx = jnp.full((LANES,), jj * CHUNK + r, jnp.int32)
                    rv = plsc.load_gather(recip_v, [bidx])
                    for c2 in range(f_half // LANES):
                        sl = pl.ds(c2 * LANES, LANES)
                        buf[r, sl] = buf[r, sl] * rv
                    return carry2
                lax.fori_loop(0, CHUNK, f_row, 0)

            def store(buf, rr, sems):
                w = [pltpu.async_copy(
                    buf, u_out.at[k - 1, c, pl.ds(rr, CHUNK)], sems[0])]
                if k < KSTEPS:
                    w.append(pltpu.async_copy(buf, U.at[pl.ds(rr, CHUNK)],
                                              sems[1]))
                    w.append(pltpu.async_copy(buf, Y.at[pl.ds(rr, CHUNK)],
                                              sems[2]))
                return w

            def f_ew(j, carry):
                j0 = 2 * j
                r0 = row0 + j0 * CHUNK
                r1 = r0 + CHUNK
                rd0 = pltpu.async_copy(Y.at[pl.ds(r0, CHUNK)], ytile, yr0)
                rd1 = pltpu.async_copy(Y.at[pl.ds(r1, CHUNK)], ytile2, yr1)
                rd0.wait()
                scale(ytile, j0)
                w0 = store(ytile, r0, yw0)
                rd1.wait()
                scale(ytile2, j0 + 1)
                w1 = store(ytile2, r1, yw1)
                for w in w0 + w1:
                    w.wait()
                return carry
            lax.fori_loop(0, nsub // 2, f_ew, 0)
            if nsub % 2:
                r0 = row0 + (nsub - 1) * CHUNK
                pltpu.sync_copy(Y.at[pl.ds(r0, CHUNK)], ytile)
                scale(ytile, nsub - 1)
                for w in store(ytile, r0, yw0):
                    w.wait()
            if k < KSTEPS:
                plsc.subcore_barrier()

    return prop


def _tc_linear(n_pad, d_in, d_out, blk):
    f_half = d_in // 2
    coef = (1.0 - ALPHA) / KSTEPS

    def body(x_ref, u_ref, deg_ref, w_ref, b_ref, o_ref):
        dg = jnp.sqrt(deg_ref[...])
        us = u_ref[...]
        ssum = us[0]
        for k in range(1, KSTEPS):
            ssum = ssum + us[k]
        s_full = jnp.concatenate([ssum[0], ssum[1]], axis=1)
        xo = ALPHA * x_ref[...] + coef * dg * s_full
        o_ref[...] = (
            jnp.dot(xo, w_ref[...], preferred_element_type=jnp.float32)
            + b_ref[...]
        )

    return pl.pallas_call(
        body,
        grid=(n_pad // blk,),
        in_specs=[
            pl.BlockSpec((blk, d_in), lambda i: (i, 0)),
            pl.BlockSpec((KSTEPS, NCORE, blk, f_half), lambda i: (0, 0, i, 0)),
            pl.BlockSpec((blk, 1), lambda i: (i, 0)),
            pl.BlockSpec((d_in, d_out), lambda i: (0, 0)),
            pl.BlockSpec((1, d_out), lambda i: (0, 0)),
        ],
        out_specs=pl.BlockSpec((blk, d_out), lambda i: (i, 0)),
        out_shape=jax.ShapeDtypeStruct((n_pad, d_out), jnp.float32),
    )


def _tc_linear_exact(n, n_pad, d_in, d_out, blk):
    f_half = d_in // 2
    coef = (1.0 - ALPHA) / KSTEPS

    def body(x_ref, u_ref, deg_ref, w_ref, b_ref, o_ref):
        dg = jnp.sqrt(deg_ref[...])
        us = u_ref[...]
        ssum = us[0]
        for kk in range(1, KSTEPS):
            ssum = ssum + us[kk]
        s_full = jnp.concatenate([ssum[0], ssum[1]], axis=1)
        xo = ALPHA * x_ref[...] + coef * dg * s_full
        o_ref[...] = (
            jnp.dot(xo, w_ref[...], preferred_element_type=jnp.float32)
            + b_ref[...]
        )

    return pl.pallas_call(
        body,
        grid=(n // blk,),
        in_specs=[
            pl.BlockSpec((blk, d_in), lambda i: (i, 0)),
            pl.BlockSpec((KSTEPS, NCORE, blk, f_half), lambda i: (0, 0, i, 0)),
            pl.BlockSpec((blk, 1), lambda i: (i, 0)),
            pl.BlockSpec((d_in, d_out), lambda i: (0, 0)),
            pl.BlockSpec((1, d_out), lambda i: (0, 0)),
        ],
        out_specs=pl.BlockSpec((blk, d_out), lambda i: (i, 0)),
        out_shape=jax.ShapeDtypeStruct((n, d_out), jnp.float32),
    )


def kernel(x, edge_index, W, b):
    n, d_in = x.shape
    d_out = W.shape[1]
    e = edge_index.shape[1]
    f_half = d_in // NCORE

    rpt = -(-n // (NSUB * CHUNK)) * CHUNK
    n_pad = NSUB * rpt
    m = -(-e // (NSUB * CHUNK * 3))
    if m % 2 == 0:
        m += 1
    chunks = 3 * m
    e_pad = NSUB * chunks * CHUNK

    src = edge_index[0]
    dst = edge_index[1]
    npad_e = e_pad - e
    if npad_e:
        fill = jnp.arange(npad_e, dtype=jnp.int32)
        src = jnp.concatenate([src, fill % n])
        dst = jnp.concatenate([dst, n + fill % (n_pad - n)])
    ei = jnp.stack([src.reshape(-1, CHUNK), dst.reshape(-1, CHUNK)], axis=1)

    deg2 = _sc_degree(n_pad, rpt, chunks)(ei)
    deg = deg2[0] + deg2[1]
    dinv = lax.rsqrt(deg)
    recip = dinv * dinv

    x2 = jnp.stack([x[:, :f_half], x[:, f_half:]])
    x2 = jnp.pad(x2, ((0, 0), (0, n_pad - n), (0, 0)))

    u_all = _sc_propagate(n_pad, f_half, rpt, chunks)(x2, ei, dinv, recip)

    if n % 1000 == 0:
        return _tc_linear_exact(n, n_pad, d_in, d_out, 1000)(
            x, u_all, deg[:, None], W, b[None, :]
        )
    x_pad = jnp.pad(x, ((0, n_pad - n), (0, 0)))
    out = _tc_linear(n_pad, d_in, d_out, 1024)(
        x_pad, u_all, deg[:, None], W, b[None, :]
    )
    return out[:n]


# speedup vs baseline: 1.0341x; 1.0341x over previous
import functools

import jax
import jax.numpy as jnp
from jax import lax
from jax.experimental import pallas as pl
from jax.experimental.pallas import tpu as pltpu
from jax.experimental.pallas import tpu_sc as plsc

ALPHA = 0.1
KSTEPS = 5
NSUB = 16
NCORE = 2
LANES = 16
CHUNK = 128

_SC_PARAMS = pltpu.CompilerParams(
    needs_layout_passes=False, use_tc_tiling_on_sc=False)


def _sc_degree(n_pad, rpt, chunks):
    mesh = plsc.VectorSubcoreMesh(core_axis_name="c", subcore_axis_name="s")

    half = -(-chunks // 2)

    @functools.partial(
        pl.kernel,
        out_type=jax.ShapeDtypeStruct((NCORE, n_pad), jnp.float32),
        mesh=mesh,
        compiler_params=_SC_PARAMS,
        scratch_types=[
            pltpu.VMEM_SHARED((n_pad,), jnp.float32),
            pltpu.VMEM((CHUNK,), jnp.int32),
            pltpu.VMEM((CHUNK,), jnp.int32),
            pltpu.VMEM((CHUNK,), jnp.float32),
            pltpu.VMEM((CHUNK,), jnp.float32),
            pltpu.SemaphoreType.DMA,
            pltpu.SemaphoreType.DMA,
        ],
    )
    def deg_kernel(ei_hbm, deg_out, DEG, didx_a, didx_b, ones_t, init_t,
                   isem_a, isem_b):
        c = lax.axis_index("c")
        s = lax.axis_index("s")
        row0 = s * rpt
        ones16 = jnp.full((LANES,), 1.0, jnp.float32)
        init16 = jnp.full((LANES,), jnp.where(c == 0, 1.0, 0.0))

        def f_ones(i, carry):
            ones_t[pl.ds(i * LANES, LANES)] = ones16
            init_t[pl.ds(i * LANES, LANES)] = init16
            return carry
        lax.fori_loop(0, CHUNK // LANES, f_ones, 0)

        def f_deginit(j, carry):
            pltpu.sync_copy(init_t, DEG.at[pl.ds(row0 + j * CHUNK, CHUNK)])
            return carry
        lax.fori_loop(0, rpt // CHUNK, f_deginit, 0)
        plsc.subcore_barrier()

        w0 = s * chunks + c * half
        hc = jnp.where(c == 0, half, chunks - half)
        wlast = s * chunks + chunks - 1
        ia = pltpu.make_async_copy(ei_hbm.at[w0, 1], didx_a, isem_a)
        ia.start()

        def f_deg(i2, carry):
            c0 = w0 + 2 * i2
            ib = pltpu.make_async_copy(
                ei_hbm.at[jnp.minimum(c0 + 1, wlast), 1], didx_b, isem_b)
            ib.start()
            ia.wait()

            @pl.when(c0 < w0 + hc)
            def _():
                pltpu.sync_copy(ones_t, DEG.at[didx_a], add=True)
            ia2 = pltpu.make_async_copy(
                ei_hbm.at[jnp.minimum(c0 + 2, wlast), 1], didx_a, isem_a)
            ia2.start()
            ib.wait()

            @pl.when(c0 + 1 < w0 + hc)
            def _():
                pltpu.sync_copy(ones_t, DEG.at[didx_b], add=True)
            return carry
        lax.fori_loop(0, (half + 1) // 2, f_deg, 0)
        ia.wait()
        plsc.subcore_barrier()
        pltpu.sync_copy(DEG.at[pl.ds(row0, rpt)],
                        deg_out.at[c, pl.ds(row0, rpt)])

    return deg_kernel


def _sc_propagate(n_pad, f_half, rpt, chunks):
    mesh = plsc.VectorSubcoreMesh(core_axis_name="c", subcore_axis_name="s")

    @functools.partial(
        pl.kernel,
        out_type=jax.ShapeDtypeStruct((KSTEPS, NCORE, n_pad, f_half),
                                      jnp.float32),
        mesh=mesh,
        compiler_params=_SC_PARAMS,
        scratch_types=[
            pltpu.VMEM_SHARED((n_pad, f_half), jnp.float32),
            pltpu.VMEM_SHARED((n_pad, f_half), jnp.float32),
            pltpu.VMEM((CHUNK, f_half), jnp.float32),
            pltpu.VMEM((CHUNK, f_half), jnp.float32),
            pltpu.VMEM((CHUNK, f_half), jnp.float32),
            pltpu.VMEM((CHUNK, f_half), jnp.float32),
            pltpu.VMEM((CHUNK, f_half), jnp.float32),
            pltpu.VMEM((2, CHUNK), jnp.int32),
            pltpu.VMEM((2, CHUNK), jnp.int32),
            pltpu.VMEM((2, CHUNK), jnp.int32),
            pltpu.VMEM((rpt,), jnp.float32),
            pltpu.VMEM((rpt,), jnp.float32),
            pltpu.SemaphoreType.DMA,
            pltpu.SemaphoreType.DMA,
            pltpu.SemaphoreType.DMA,
            pltpu.SemaphoreType.DMA,
            pltpu.SemaphoreType.DMA,
            pltpu.SemaphoreType.DMA,
            pltpu.SemaphoreType.DMA,
            pltpu.SemaphoreType.DMA,
            pltpu.SemaphoreType.DMA,
            pltpu.SemaphoreType.DMA,
        ],
    )
    def prop(x2_hbm, ei_hbm, dinv_hbm, recip_hbm, u_out,
             U, Y, ytile, ytile2, rows_0, rows_1, rows_2, idx_0, idx_1, idx_2,
             dinv_v, recip_v, gsem_0, gsem_1, gsem_2, ssem_0, ssem_1, ssem_2,
             yr0, yr1, yw0, yw1):
        c = lax.axis_index("c")
        s = lax.axis_index("s")
        row0 = s * rpt
        ch0 = s * chunks
        nsub = rpt // CHUNK

        pltpu.sync_copy(dinv_hbm.at[pl.ds(row0, rpt)], dinv_v)
        pltpu.sync_copy(recip_hbm.at[pl.ds(row0, rpt)], recip_v)

        def f_x(j, carry):
            r0 = row0 + j * CHUNK
            pltpu.sync_copy(x2_hbm.at[c, pl.ds(r0, CHUNK)], ytile)

            def f_row(r, carry2):
                idx = jnp.full((LANES,), j * CHUNK + r, jnp.int32)
                dv = plsc.load_gather(dinv_v, [idx])
                for c2 in range(f_half // LANES):
                    sl = pl.ds(c2 * LANES, LANES)
                    ytile[r, sl] = ytile[r, sl] * dv
                return carry2
            lax.fori_loop(0, CHUNK, f_row, 0)
            pltpu.sync_copy(ytile, U.at[pl.ds(r0, CHUNK)])
            pltpu.sync_copy(ytile, Y.at[pl.ds(r0, CHUNK)])
            return carry
        lax.fori_loop(0, nsub, f_x, 0)
        plsc.subcore_barrier()


        rows = (rows_0, rows_1, rows_2)
        idx = (idx_0, idx_1, idx_2)
        gsem = (gsem_0, gsem_1, gsem_2)
        ssem = (ssem_0, ssem_1, ssem_2)
        gd = tuple(pltpu.make_async_copy(U.at[idx[j].at[0]], rows[j], gsem[j])
                   for j in range(3))
        clast = ch0 + chunks - 1

        for k in range(1, KSTEPS + 1):
            pltpu.sync_copy(ei_hbm.at[ch0], idx_0)
            gd[0].start()
            pltpu.sync_copy(ei_hbm.at[ch0 + 1], idx_1)
            gd[1].start()

            def f_tri(i3, carry):
                c0 = ch0 + 3 * i3
                scat = []
                for j in range(3):
                    gd[j].wait()
                    scat.append(pltpu.async_copy(
                        rows[j], Y.at[idx[j].at[1]], ssem[j], add=True))
                    if j > 0:
                        scat[j - 1].wait()
                        jp = j - 1
                    else:
                        jp = 2
                    cn = jnp.minimum(c0 + 2 + j, clast)
                    pltpu.sync_copy(ei_hbm.at[cn], idx[jp])
                    gd[jp].start()
                scat[2].wait()
                return carry
            lax.fori_loop(0, chunks // 3, f_tri, 0)
            gd[0].wait()
            gd[1].wait()
            plsc.subcore_barrier()

            def scale(buf, jj):
                def f_row(r, carry2):
                    bidx = jnp.full((LANES,), jj * CHUNK + r, jnp.int32)
                    rv = plsc.load_gather(recip_v, [bidx])
                    for c2 in range(f_half // LANES):
                        sl = pl.ds(c2 * LANES, LANES)
                        buf[r, sl] = buf[r, sl] * rv
                    return carry2
                lax.fori_loop(0, CHUNK, f_row, 0)

            def store(buf, rr):
                pltpu.sync_copy(buf, u_out.at[k - 1, c, pl.ds(rr, CHUNK)])
                if k < KSTEPS:
                    pltpu.sync_copy(buf, U.at[pl.ds(rr, CHUNK)])
                    pltpu.sync_copy(buf, Y.at[pl.ds(rr, CHUNK)])

            def f_ew(j, carry):
                j0 = 2 * j
                r0 = row0 + j0 * CHUNK
                r1 = r0 + CHUNK
                rd0 = pltpu.async_copy(Y.at[pl.ds(r0, CHUNK)], ytile, yr0)
                rd1 = pltpu.async_copy(Y.at[pl.ds(r1, CHUNK)], ytile2, yr1)
                rd0.wait()
                scale(ytile, j0)
                store(ytile, r0)
                rd1.wait()
                scale(ytile2, j0 + 1)
                store(ytile2, r1)
                return carry
            lax.fori_loop(0, nsub // 2, f_ew, 0)
            if nsub % 2:
                r0 = row0 + (nsub - 1) * CHUNK
                pltpu.sync_copy(Y.at[pl.ds(r0, CHUNK)], ytile)
                scale(ytile, nsub - 1)
                store(ytile, r0)
            if k < KSTEPS:
                plsc.subcore_barrier()

    return prop


def _tc_linear(n_pad, d_in, d_out, blk):
    f_half = d_in // 2
    coef = (1.0 - ALPHA) / KSTEPS

    def body(x_ref, u_ref, deg_ref, w_ref, b_ref, o_ref):
        dg = jnp.sqrt(deg_ref[...])
        us = u_ref[...]
        ssum = us[0]
        for k in range(1, KSTEPS):
            ssum = ssum + us[k]
        s_full = jnp.concatenate([ssum[0], ssum[1]], axis=1)
        xo = ALPHA * x_ref[...] + coef * dg * s_full
        o_ref[...] = (
            jnp.dot(xo, w_ref[...], preferred_element_type=jnp.float32)
            + b_ref[...]
        )

    return pl.pallas_call(
        body,
        grid=(n_pad // blk,),
        in_specs=[
            pl.BlockSpec((blk, d_in), lambda i: (i, 0)),
            pl.BlockSpec((KSTEPS, NCORE, blk, f_half), lambda i: (0, 0, i, 0)),
            pl.BlockSpec((blk, 1), lambda i: (i, 0)),
            pl.BlockSpec((d_in, d_out), lambda i: (0, 0)),
            pl.BlockSpec((1, d_out), lambda i: (0, 0)),
        ],
        out_specs=pl.BlockSpec((blk, d_out), lambda i: (i, 0)),
        out_shape=jax.ShapeDtypeStruct((n_pad, d_out), jnp.float32),
    )


def _tc_linear_exact(n, n_pad, d_in, d_out, blk):
    f_half = d_in // 2
    coef = (1.0 - ALPHA) / KSTEPS

    def body(x_ref, u_ref, deg_ref, w_ref, b_ref, o_ref):
        dg = jnp.sqrt(deg_ref[...])
        us = u_ref[...]
        ssum = us[0]
        for kk in range(1, KSTEPS):
            ssum = ssum + us[kk]
        s_full = jnp.concatenate([ssum[0], ssum[1]], axis=1)
        xo = ALPHA * x_ref[...] + coef * dg * s_full
        o_ref[...] = (
            jnp.dot(xo, w_ref[...], preferred_element_type=jnp.float32)
            + b_ref[...]
        )

    return pl.pallas_call(
        body,
        grid=(n // blk,),
        in_specs=[
            pl.BlockSpec((blk, d_in), lambda i: (i, 0)),
            pl.BlockSpec((KSTEPS, NCORE, blk, f_half), lambda i: (0, 0, i, 0)),
            pl.BlockSpec((blk, 1), lambda i: (i, 0)),
            pl.BlockSpec((d_in, d_out), lambda i: (0, 0)),
            pl.BlockSpec((1, d_out), lambda i: (0, 0)),
        ],
        out_specs=pl.BlockSpec((blk, d_out), lambda i: (i, 0)),
        out_shape=jax.ShapeDtypeStruct((n, d_out), jnp.float32),
    )


def kernel(x, edge_index, W, b):
    n, d_in = x.shape
    d_out = W.shape[1]
    e = edge_index.shape[1]
    f_half = d_in // NCORE

    rpt = -(-n // (NSUB * CHUNK)) * CHUNK
    n_pad = NSUB * rpt
    m = -(-e // (NSUB * CHUNK * 3))
    if m % 2 == 0:
        m += 1
    chunks = 3 * m
    e_pad = NSUB * chunks * CHUNK

    src = edge_index[0]
    dst = edge_index[1]
    npad_e = e_pad - e
    if npad_e:
        fill = jnp.arange(npad_e, dtype=jnp.int32)
        src = jnp.concatenate([src, fill % n])
        dst = jnp.concatenate([dst, n + fill % (n_pad - n)])
    ei = jnp.stack([src.reshape(-1, CHUNK), dst.reshape(-1, CHUNK)], axis=1)

    deg2 = _sc_degree(n_pad, rpt, chunks)(ei)
    deg = deg2[0] + deg2[1]
    dinv = lax.rsqrt(deg)
    recip = dinv * dinv

    x2 = jnp.stack([x[:, :f_half], x[:, f_half:]])
    x2 = jnp.pad(x2, ((0, 0), (0, n_pad - n), (0, 0)))

    u_all = _sc_propagate(n_pad, f_half, rpt, chunks)(x2, ei, dinv, recip)

    if n % 1000 == 0:
        return _tc_linear_exact(n, n_pad, d_in, d_out, 1000)(
            x, u_all, deg[:, None], W, b[None, :]
        )
    x_pad = jnp.pad(x, ((0, n_pad - n), (0, 0)))
    out = _tc_linear(n_pad, d_in, d_out, 1024)(
        x_pad, u_all, deg[:, None], W, b[None, :]
    )
    return out[:n]


# speedup vs baseline: 1.0568x; 1.0220x over previous
import functools

import jax
import jax.numpy as jnp
from jax import lax
from jax.experimental import pallas as pl
from jax.experimental.pallas import tpu as pltpu
from jax.experimental.pallas import tpu_sc as plsc

ALPHA = 0.1
KSTEPS = 5
NSUB = 16
NCORE = 2
LANES = 16
CHUNK = 128

_SC_PARAMS = pltpu.CompilerParams(
    needs_layout_passes=False, use_tc_tiling_on_sc=False)


def _sc_degree(n_pad, rpt, chunks):
    mesh = plsc.VectorSubcoreMesh(core_axis_name="c", subcore_axis_name="s")

    half = -(-chunks // 2)

    @functools.partial(
        pl.kernel,
        out_type=jax.ShapeDtypeStruct((NCORE, n_pad), jnp.float32),
        mesh=mesh,
        compiler_params=_SC_PARAMS,
        scratch_types=[
            pltpu.VMEM_SHARED((n_pad,), jnp.float32),
            pltpu.VMEM((CHUNK,), jnp.int32),
            pltpu.VMEM((CHUNK,), jnp.int32),
            pltpu.VMEM((CHUNK,), jnp.float32),
            pltpu.VMEM((CHUNK,), jnp.float32),
            pltpu.SemaphoreType.DMA,
            pltpu.SemaphoreType.DMA,
        ],
    )
    def deg_kernel(ei_hbm, deg_out, DEG, didx_a, didx_b, ones_t, init_t,
                   isem_a, isem_b):
        c = lax.axis_index("c")
        s = lax.axis_index("s")
        row0 = s * rpt
        ones16 = jnp.full((LANES,), 1.0, jnp.float32)
        init16 = jnp.full((LANES,), jnp.where(c == 0, 1.0, 0.0))

        def f_ones(i, carry):
            ones_t[pl.ds(i * LANES, LANES)] = ones16
            init_t[pl.ds(i * LANES, LANES)] = init16
            return carry
        lax.fori_loop(0, CHUNK // LANES, f_ones, 0)

        def f_deginit(j, carry):
            pltpu.sync_copy(init_t, DEG.at[pl.ds(row0 + j * CHUNK, CHUNK)])
            return carry
        lax.fori_loop(0, rpt // CHUNK, f_deginit, 0)
        plsc.subcore_barrier()

        w0 = s * chunks + c * half
        hc = jnp.where(c == 0, half, chunks - half)
        wlast = s * chunks + chunks - 1
        ia = pltpu.make_async_copy(ei_hbm.at[w0, 1], didx_a, isem_a)
        ia.start()

        def f_deg(i2, carry):
            c0 = w0 + 2 * i2
            ib = pltpu.make_async_copy(
                ei_hbm.at[jnp.minimum(c0 + 1, wlast), 1], didx_b, isem_b)
            ib.start()
            ia.wait()

            @pl.when(c0 < w0 + hc)
            def _():
                pltpu.sync_copy(ones_t, DEG.at[didx_a], add=True)
            ia2 = pltpu.make_async_copy(
                ei_hbm.at[jnp.minimum(c0 + 2, wlast), 1], didx_a, isem_a)
            ia2.start()
            ib.wait()

            @pl.when(c0 + 1 < w0 + hc)
            def _():
                pltpu.sync_copy(ones_t, DEG.at[didx_b], add=True)
            return carry
        lax.fori_loop(0, (half + 1) // 2, f_deg, 0)
        ia.wait()
        plsc.subcore_barrier()
        pltpu.sync_copy(DEG.at[pl.ds(row0, rpt)],
                        deg_out.at[c, pl.ds(row0, rpt)])

    return deg_kernel


def _sc_propagate(n_pad, f_half, rpt, chunks):
    mesh = plsc.VectorSubcoreMesh(core_axis_name="c", subcore_axis_name="s")

    @functools.partial(
        pl.kernel,
        out_type=jax.ShapeDtypeStruct((KSTEPS, NCORE, n_pad, f_half),
                                      jnp.float32),
        mesh=mesh,
        compiler_params=_SC_PARAMS,
        scratch_types=[
            pltpu.VMEM_SHARED((n_pad, f_half), jnp.float32),
            pltpu.VMEM_SHARED((n_pad, f_half), jnp.float32),
            pltpu.VMEM((CHUNK, f_half), jnp.float32),
            pltpu.VMEM((CHUNK, f_half), jnp.float32),
            pltpu.VMEM((CHUNK, f_half), jnp.float32),
            pltpu.VMEM((CHUNK, f_half), jnp.float32),
            pltpu.VMEM((CHUNK, f_half), jnp.float32),
            pltpu.VMEM((2, CHUNK), jnp.int32),
            pltpu.VMEM((2, CHUNK), jnp.int32),
            pltpu.VMEM((2, CHUNK), jnp.int32),
            pltpu.VMEM((2, CHUNK), jnp.int32),
            pltpu.VMEM((rpt,), jnp.float32),
            pltpu.VMEM((rpt,), jnp.float32),
            pltpu.SemaphoreType.DMA,
            pltpu.SemaphoreType.DMA,
            pltpu.SemaphoreType.DMA,
            pltpu.SemaphoreType.DMA,
            pltpu.SemaphoreType.DMA,
            pltpu.SemaphoreType.DMA,
            pltpu.SemaphoreType.DMA,
            pltpu.SemaphoreType.DMA,
        ],
    )
    def prop(x2_hbm, ei_hbm, dinv_hbm, recip_hbm, u_out,
             U, Y, ytile, rows_0, rows_1, rows_2, rows_3,
             idx_0, idx_1, idx_2, idx_3, dinv_v, recip_v,
             gsem_0, gsem_1, gsem_2, gsem_3, ssem_0, ssem_1, ssem_2, ssem_3):
        c = lax.axis_index("c")
        s = lax.axis_index("s")
        row0 = s * rpt
        ch0 = s * chunks
        nsub = rpt // CHUNK

        pltpu.sync_copy(dinv_hbm.at[pl.ds(row0, rpt)], dinv_v)
        pltpu.sync_copy(recip_hbm.at[pl.ds(row0, rpt)], recip_v)

        def f_x(j, carry):
            r0 = row0 + j * CHUNK
            pltpu.sync_copy(x2_hbm.at[c, pl.ds(r0, CHUNK)], ytile)

            def f_row(r, carry2):
                idx = jnp.full((LANES,), j * CHUNK + r, jnp.int32)
                dv = plsc.load_gather(dinv_v, [idx])
                for c2 in range(f_half // LANES):
                    sl = pl.ds(c2 * LANES, LANES)
                    ytile[r, sl] = ytile[r, sl] * dv
                return carry2
            lax.fori_loop(0, CHUNK, f_row, 0)
            pltpu.sync_copy(ytile, U.at[pl.ds(r0, CHUNK)])
            pltpu.sync_copy(ytile, Y.at[pl.ds(r0, CHUNK)])
            return carry
        lax.fori_loop(0, nsub, f_x, 0)
        plsc.subcore_barrier()


        rows = (rows_0, rows_1, rows_2, rows_3)
        idx = (idx_0, idx_1, idx_2, idx_3)
        gsem = (gsem_0, gsem_1, gsem_2, gsem_3)
        ssem = (ssem_0, ssem_1, ssem_2, ssem_3)
        gd = tuple(pltpu.make_async_copy(U.at[idx[j].at[0]], rows[j], gsem[j])
                   for j in range(4))
        clast = ch0 + chunks - 1

        for k in range(1, KSTEPS + 1):
            pltpu.sync_copy(ei_hbm.at[ch0], idx_0)
            gd[0].start()
            pltpu.sync_copy(ei_hbm.at[ch0 + 1], idx_1)
            gd[1].start()
            pltpu.sync_copy(ei_hbm.at[ch0 + 2], idx_2)
            gd[2].start()

            def f_quad(i4, carry):
                c0 = ch0 + 4 * i4
                scat = []
                for j in range(4):
                    gd[j].wait()
                    scat.append(pltpu.async_copy(
                        rows[j], Y.at[idx[j].at[1]], ssem[j], add=True))
                    if j > 0:
                        scat[j - 1].wait()
                        jp = j - 1
                    else:
                        jp = 3
                    cn = jnp.minimum(c0 + 3 + j, clast)
                    pltpu.sync_copy(ei_hbm.at[cn], idx[jp])
                    gd[jp].start()
                scat[3].wait()
                return carry
            lax.fori_loop(0, chunks // 4, f_quad, 0)
            gd[0].wait()
            gd[1].wait()
            gd[2].wait()
            plsc.subcore_barrier()

            def scale(buf, jj):
                def f_row(r, carry2):
                    bidx = jnp.full((LANES,), jj * CHUNK + r, jnp.int32)
                    rv = plsc.load_gather(recip_v, [bidx])
                    for c2 in range(f_half // LANES):
                        sl = pl.ds(c2 * LANES, LANES)
                        buf[r, sl] = buf[r, sl] * rv
                    return carry2
                lax.fori_loop(0, CHUNK, f_row, 0)

            def f_ew(j, carry):
                r0 = row0 + j * CHUNK
                pltpu.sync_copy(Y.at[pl.ds(r0, CHUNK)], ytile)
                scale(ytile, j)
                pltpu.sync_copy(ytile, u_out.at[k - 1, c, pl.ds(r0, CHUNK)])
                if k < KSTEPS:
                    pltpu.sync_copy(ytile, U.at[pl.ds(r0, CHUNK)])
                    pltpu.sync_copy(ytile, Y.at[pl.ds(r0, CHUNK)])
                return carry
            lax.fori_loop(0, nsub, f_ew, 0)
            if k < KSTEPS:
                plsc.subcore_barrier()

    return prop


def _tc_linear(n_pad, d_in, d_out, blk):
    f_half = d_in // 2
    coef = (1.0 - ALPHA) / KSTEPS

    def body(x_ref, u_ref, deg_ref, w_ref, b_ref, o_ref):
        dg = jnp.sqrt(deg_ref[...])
        us = u_ref[...]
        ssum = us[0]
        for k in range(1, KSTEPS):
            ssum = ssum + us[k]
        s_full = jnp.concatenate([ssum[0], ssum[1]], axis=1)
        xo = ALPHA * x_ref[...] + coef * dg * s_full
        o_ref[...] = (
            jnp.dot(xo, w_ref[...], preferred_element_type=jnp.float32)
            + b_ref[...]
        )

    return pl.pallas_call(
        body,
        grid=(n_pad // blk,),
        in_specs=[
            pl.BlockSpec((blk, d_in), lambda i: (i, 0)),
            pl.BlockSpec((KSTEPS, NCORE, blk, f_half), lambda i: (0, 0, i, 0)),
            pl.BlockSpec((blk, 1), lambda i: (i, 0)),
            pl.BlockSpec((d_in, d_out), lambda i: (0, 0)),
            pl.BlockSpec((1, d_out), lambda i: (0, 0)),
        ],
        out_specs=pl.BlockSpec((blk, d_out), lambda i: (i, 0)),
        out_shape=jax.ShapeDtypeStruct((n_pad, d_out), jnp.float32),
    )


def _tc_linear_exact(n, n_pad, d_in, d_out, blk):
    f_half = d_in // 2
    coef = (1.0 - ALPHA) / KSTEPS

    def body(x_ref, u_ref, deg_ref, w_ref, b_ref, o_ref):
        dg = jnp.sqrt(deg_ref[...])
        us = u_ref[...]
        ssum = us[0]
        for kk in range(1, KSTEPS):
            ssum = ssum + us[kk]
        s_full = jnp.concatenate([ssum[0], ssum[1]], axis=1)
        xo = ALPHA * x_ref[...] + coef * dg * s_full
        o_ref[...] = (
            jnp.dot(xo, w_ref[...], preferred_element_type=jnp.float32)
            + b_ref[...]
        )

    return pl.pallas_call(
        body,
        grid=(n // blk,),
        in_specs=[
            pl.BlockSpec((blk, d_in), lambda i: (i, 0)),
            pl.BlockSpec((KSTEPS, NCORE, blk, f_half), lambda i: (0, 0, i, 0)),
            pl.BlockSpec((blk, 1), lambda i: (i, 0)),
            pl.BlockSpec((d_in, d_out), lambda i: (0, 0)),
            pl.BlockSpec((1, d_out), lambda i: (0, 0)),
        ],
        out_specs=pl.BlockSpec((blk, d_out), lambda i: (i, 0)),
        out_shape=jax.ShapeDtypeStruct((n, d_out), jnp.float32),
    )


def kernel(x, edge_index, W, b):
    n, d_in = x.shape
    d_out = W.shape[1]
    e = edge_index.shape[1]
    f_half = d_in // NCORE

    rpt = -(-n // (NSUB * CHUNK)) * CHUNK
    n_pad = NSUB * rpt
    chunks = 4 * -(-e // (NSUB * CHUNK * 4))
    e_pad = NSUB * chunks * CHUNK

    src = edge_index[0]
    dst = edge_index[1]
    npad_e = e_pad - e
    if npad_e:
        fill = jnp.arange(npad_e, dtype=jnp.int32)
        src = jnp.concatenate([src, fill % n])
        dst = jnp.concatenate([dst, n + fill % (n_pad - n)])
    ei = jnp.stack([src.reshape(-1, CHUNK), dst.reshape(-1, CHUNK)], axis=1)

    deg2 = _sc_degree(n_pad, rpt, chunks)(ei)
    deg = deg2[0] + deg2[1]
    dinv = lax.rsqrt(deg)
    recip = dinv * dinv

    x2 = jnp.stack([x[:, :f_half], x[:, f_half:]])
    x2 = jnp.pad(x2, ((0, 0), (0, n_pad - n), (0, 0)))

    u_all = _sc_propagate(n_pad, f_half, rpt, chunks)(x2, ei, dinv, recip)

    if n % 1000 == 0:
        return _tc_linear_exact(n, n_pad, d_in, d_out, 1000)(
            x, u_all, deg[:, None], W, b[None, :]
        )
    x_pad = jnp.pad(x, ((0, n_pad - n), (0, 0)))
    out = _tc_linear(n_pad, d_in, d_out, 1024)(
        x_pad, u_all, deg[:, None], W, b[None, :]
    )
    return out[:n]
